# baseline (speedup 1.0000x reference)
import functools

import jax
import jax.numpy as jnp
from jax.experimental import pallas as pl
from jax.experimental.pallas import tpu as pltpu

_LANE = 128
_TARGET_BLOCK_BYTES = 2 * 2**20
_MAX_BLOCK_BYTES = 8 * 2**20


def _ceil_to(x, m):
    return -(-x // m) * m


def _maybe_pad(a, shape):
    pads = tuple((0, s - d) for d, s in zip(a.shape, shape))
    if any(hi for _, hi in pads):
        return jnp.pad(a, pads)
    return a


def _soca_block_kernel(alpha_ref, x_ref, w1s_ref, b1_ref, w2t_ref, b2_ref,
                       o_ref):
    x = x_ref[...]

    pooled = jnp.sum(x.astype(jnp.float32), axis=-1)

    h = jnp.dot(pooled, w1s_ref[...],
                preferred_element_type=jnp.float32) + b1_ref[...]
    a = alpha_ref[0]
    h = jnp.maximum(h, 0.0) + a * jnp.minimum(h, 0.0)
    z = jnp.dot(h, w2t_ref[...],
                preferred_element_type=jnp.float32) + b2_ref[...]
    s = jax.nn.sigmoid(z).astype(o_ref.dtype)

    o_ref[...] = x * s[:, :, None]


def kernel(x, w1, b1, w2, b2, alpha):
    B, C, H, W = x.shape
    hidden = w1.shape[0]
    HW = H * W
    dtype = x.dtype
    itemsize = jnp.dtype(dtype).itemsize

    sub = {4: 8, 2: 16, 1: 32}.get(itemsize, 8)
    C_p = _ceil_to(C, sub)
    HW_p = _ceil_to(HW, _LANE)
    row_bytes = C_p * HW_p * itemsize

    bt = max(1, min(B, _TARGET_BLOCK_BYTES // max(1, row_bytes)))
    if row_bytes > _MAX_BLOCK_BYTES:
        bt = 1
    nb = -(-B // bt)
    B_p = nb * bt

    inv_hw = 1.0 / float(HW)
    w1s = (_maybe_pad(w1, (hidden, C_p)).T * inv_hw).astype(jnp.float32)
    w2t = _maybe_pad(w2, (C_p, hidden)).T.astype(jnp.float32)
    b1r = b1.astype(jnp.float32).reshape(1, hidden)
    b2r = _maybe_pad(b2.astype(jnp.float32), (C_p,)).reshape(1, C_p)
    alpha_f = alpha.astype(jnp.float32).reshape(1)

    x3 = _maybe_pad(x.reshape(B, C, HW), (B_p, C_p, HW_p))
    block_bytes = bt * row_bytes
    vmem = int(min(100 * 2**20, 4 * block_bytes + 6 * 2**20))

    out = pl.pallas_call(
        _soca_block_kernel,
        out_shape=jax.ShapeDtypeStruct((B_p, C_p, HW_p), dtype),
        grid=(nb,),
        in_specs=[
            pl.BlockSpec(memory_space=pltpu.MemorySpace.SMEM),
            pl.BlockSpec((bt, C_p, HW_p), lambda b: (b, 0, 0)),
            pl.BlockSpec((C_p, hidden), lambda b: (0, 0)),
            pl.BlockSpec((1, hidden), lambda b: (0, 0)),
            pl.BlockSpec((hidden, C_p), lambda b: (0, 0)),
            pl.BlockSpec((1, C_p), lambda b: (0, 0)),
        ],
        out_specs=pl.BlockSpec((bt, C_p, HW_p), lambda b: (b, 0, 0)),
        compiler_params=pltpu.CompilerParams(
            dimension_semantics=("parallel",),
            vmem_limit_bytes=vmem),
    )(alpha_f, x3, w1s, b1r, w2t, b2r)

    if (B_p, C_p, HW_p) != (B, C, HW):
        out = out[:B, :C, :HW]
    return out.reshape(B, C, H, W)


# speedup vs baseline: 2.9309x; 2.9309x over previous
import functools

import jax
import jax.numpy as jnp
from jax.experimental import pallas as pl
from jax.experimental.pallas import tpu as pltpu

_LANE = 128
_SUBLANE = 8
_TARGET_BLOCK_BYTES = 4 * 2**20


def _soca4d_kernel(alpha_ref, x_ref, w1s_ref, b1_ref, w2t_ref, b2_ref, o_ref):
    x = x_ref[...]

    t = jnp.sum(x.astype(jnp.float32), axis=2)
    pooled = jnp.sum(t, axis=-1)

    h = jnp.dot(pooled, w1s_ref[...],
                preferred_element_type=jnp.float32) + b1_ref[...]
    a = alpha_ref[0]
    h = jnp.maximum(h, 0.0) + a * jnp.minimum(h, 0.0)
    z = jnp.dot(h, w2t_ref[...],
                preferred_element_type=jnp.float32) + b2_ref[...]
    s = jax.nn.sigmoid(z).astype(o_ref.dtype)

    s4 = jax.lax.broadcast_in_dim(s, x.shape, (0, 1))
    o_ref[...] = x * s4


def kernel(x, w1, b1, w2, b2, alpha):
    B, C, H, W = x.shape
    hidden = w1.shape[0]
    dtype = x.dtype
    itemsize = jnp.dtype(dtype).itemsize

    W_pad = -(-W // _LANE) * _LANE
    H_pad = -(-H // _SUBLANE) * _SUBLANE
    row_bytes = C * H_pad * W_pad * itemsize

    bt = 1
    for cand in range(1, B + 1):
        if B % cand == 0 and cand * row_bytes <= _TARGET_BLOCK_BYTES:
            bt = cand
    nb = B // bt

    inv_hw = 1.0 / float(H * W)
    w1s = (w1.T * inv_hw).astype(jnp.float32)
    w2t = w2.T.astype(jnp.float32)
    b1r = b1.astype(jnp.float32).reshape(1, hidden)
    b2r = b2.astype(jnp.float32).reshape(1, C)
    alpha_f = alpha.astype(jnp.float32).reshape(1)

    block_bytes = bt * row_bytes
    vmem = int(min(100 * 2**20, 4 * block_bytes + 6 * 2**20))

    return pl.pallas_call(
        _soca4d_kernel,
        out_shape=jax.ShapeDtypeStruct((B, C, H, W), dtype),
        grid=(nb,),
        in_specs=[
            pl.BlockSpec(memory_space=pltpu.MemorySpace.SMEM),
            pl.BlockSpec((bt, C, H, W), lambda b: (b, 0, 0, 0)),
            pl.BlockSpec((C, hidden), lambda b: (0, 0)),
            pl.BlockSpec((1, hidden), lambda b: (0, 0)),
            pl.BlockSpec((hidden, C), lambda b: (0, 0)),
            pl.BlockSpec((1, C), lambda b: (0, 0)),
        ],
        out_specs=pl.BlockSpec((bt, C, H, W), lambda b: (b, 0, 0, 0)),
        compiler_params=pltpu.CompilerParams(
            dimension_semantics=("parallel",),
            vmem_limit_bytes=vmem),
    )(alpha_f, x, w1s, b1r, w2t, b2r)


# speedup vs baseline: 3.0356x; 1.0357x over previous
import functools

import jax
import jax.numpy as jnp
from jax.experimental import pallas as pl
from jax.experimental.pallas import tpu as pltpu

_LANE = 128
_SUBLANE = 8
_TARGET_BLOCK_BYTES = 8 * 2**20


def _soca4d_kernel(alpha_ref, x_ref, w1s_ref, b1_ref, w2t_ref, b2_ref, o_ref):
    x = x_ref[...]

    t = jnp.sum(x.astype(jnp.float32), axis=2)
    pooled = jnp.sum(t, axis=-1)

    h = jnp.dot(pooled, w1s_ref[...],
                preferred_element_type=jnp.float32) + b1_ref[...]
    a = alpha_ref[0]
    h = jnp.maximum(h, 0.0) + a * jnp.minimum(h, 0.0)
    z = jnp.dot(h, w2t_ref[...],
                preferred_element_type=jnp.float32) + b2_ref[...]
    s = jax.nn.sigmoid(z).astype(o_ref.dtype)

    s4 = jax.lax.broadcast_in_dim(s, x.shape, (0, 1))
    o_ref[...] = x * s4


def kernel(x, w1, b1, w2, b2, alpha):
    B, C, H, W = x.shape
    hidden = w1.shape[0]
    dtype = x.dtype
    itemsize = jnp.dtype(dtype).itemsize

    W_pad = -(-W // _LANE) * _LANE
    H_pad = -(-H // _SUBLANE) * _SUBLANE
    row_bytes = C * H_pad * W_pad * itemsize

    bt = 1
    for cand in range(1, B + 1):
        if B % cand == 0 and cand * row_bytes <= _TARGET_BLOCK_BYTES:
            bt = cand
    nb = B // bt

    inv_hw = 1.0 / float(H * W)
    w1s = (w1.T * inv_hw).astype(jnp.float32)
    w2t = w2.T.astype(jnp.float32)
    b1r = b1.astype(jnp.float32).reshape(1, hidden)
    b2r = b2.astype(jnp.float32).reshape(1, C)
    alpha_f = alpha.astype(jnp.float32).reshape(1)

    block_bytes = bt * row_bytes
    vmem = int(min(100 * 2**20, 4 * block_bytes + 6 * 2**20))

    return pl.pallas_call(
        _soca4d_kernel,
        out_shape=jax.ShapeDtypeStruct((B, C, H, W), dtype),
        grid=(nb,),
        in_specs=[
            pl.BlockSpec(memory_space=pltpu.MemorySpace.SMEM),
            pl.BlockSpec((bt, C, H, W), lambda b: (b, 0, 0, 0)),
            pl.BlockSpec((C, hidden), lambda b: (0, 0)),
            pl.BlockSpec((1, hidden), lambda b: (0, 0)),
            pl.BlockSpec((hidden, C), lambda b: (0, 0)),
            pl.BlockSpec((1, C), lambda b: (0, 0)),
        ],
        out_specs=pl.BlockSpec((bt, C, H, W), lambda b: (b, 0, 0, 0)),
        compiler_params=pltpu.CompilerParams(
            dimension_semantics=("parallel",),
            vmem_limit_bytes=vmem),
    )(alpha_f, x, w1s, b1r, w2t, b2r)
